# Initial kernel scaffold; baseline (speedup 1.0000x reference)
import functools

import jax
import jax.numpy as jnp
from jax import lax
from jax.experimental import pallas as pl
from jax.experimental.pallas import tpu as pltpu
from jax.experimental.pallas import tpu_sc as plsc

N = 50000
E = 800000
G = 128
EPS = 1e-5

NP = 50048
NP2 = 51200
NB = 391

NCORE = 2
NSUB = 16
NW = NCORE * NSUB
SLICE = NP // NSUB

EPAD = 819200
ROWS = EPAD // 128
RPW = ROWS // NW
RPS = ROWS // NSUB
CH = 8

C_POOL = 512

_mesh = plsc.VectorSubcoreMesh(
    core_axis_name="c", subcore_axis_name="s", num_cores=NCORE,
    num_subcores=NSUB)

_f32 = jnp.float32
_i32 = jnp.int32


@functools.partial(
    pl.kernel,
    out_type=jax.ShapeDtypeStruct((NCORE, NP, 16), _f32),
    mesh=_mesh,
    scratch_types=[
        pltpu.VMEM((CH, 128), _i32),
        pltpu.VMEM((128, 16), _f32),
        pltpu.VMEM_SHARED((NP, 16), _f32),
        pltpu.SemaphoreType.DMA,
    ],
)
def _deg_sc(dst_hbm, z_hbm, out_hbm, idx_v, ones_v, acc_sh, sem):
    c = lax.axis_index("c")
    s = lax.axis_index("s")
    w = c * NSUB + s

    @pl.loop(0, 128)
    def _(r):
        ones_v[pl.ds(r, 1), :] = jnp.ones((1, 16), _f32)

    pltpu.sync_copy(z_hbm.at[pl.ds(s * SLICE, SLICE)],
                    acc_sh.at[pl.ds(s * SLICE, SLICE)])
    plsc.subcore_barrier()

    @pl.loop(0, RPW // CH)
    def _(gp):
        base = w * RPW + gp * CH
        pltpu.sync_copy(dst_hbm.at[pl.ds(base, CH)], idx_v)
        cps = [pltpu.async_copy(ones_v, acc_sh.at[idx_v.at[j]], sem, add=True)
               for j in range(CH)]
        for cp in cps:
            cp.wait()

    plsc.subcore_barrier()
    pltpu.sync_copy(acc_sh.at[pl.ds(s * SLICE, SLICE)],
                    out_hbm.at[c, pl.ds(s * SLICE, SLICE)])


@functools.partial(
    pl.kernel,
    out_type=jax.ShapeDtypeStruct((NCORE, NP, 16), _f32),
    mesh=_mesh,
    scratch_types=[
        pltpu.VMEM((CH, 128), _i32),
        pltpu.VMEM((CH, 128), _i32),
        pltpu.VMEM((CH, 128, 16), _f32),
        pltpu.VMEM_SHARED((NP, 16), _f32),
        pltpu.SemaphoreType.DMA,
        pltpu.SemaphoreType.DMA,
    ],
)
def _conv1_sc(src_hbm, dst_hbm, u_hbm, z_hbm, out_hbm,
              src_v, dst_v, rows_v, acc_sh, sem_g, sem_s):
    c = lax.axis_index("c")
    s = lax.axis_index("s")
    w = c * NSUB + s

    pltpu.sync_copy(z_hbm.at[pl.ds(s * SLICE, SLICE)],
                    acc_sh.at[pl.ds(s * SLICE, SLICE)])
    plsc.subcore_barrier()

    @pl.loop(0, RPW // CH)
    def _(gp):
        base = w * RPW + gp * CH
        pltpu.sync_copy(src_hbm.at[pl.ds(base, CH)], src_v)
        pltpu.sync_copy(dst_hbm.at[pl.ds(base, CH)], dst_v)
        gcps = [pltpu.async_copy(u_hbm.at[src_v.at[j]], rows_v.at[j], sem_g)
                for j in range(CH)]
        for cp in gcps:
            cp.wait()
        scps = [pltpu.async_copy(rows_v.at[j], acc_sh.at[dst_v.at[j]], sem_s,
                                 add=True)
                for j in range(CH)]
        for cp in scps:
            cp.wait()

    plsc.subcore_barrier()
    pltpu.sync_copy(acc_sh.at[pl.ds(s * SLICE, SLICE)],
                    out_hbm.at[c, pl.ds(s * SLICE, SLICE)])


@functools.partial(
    pl.kernel,
    out_type=jax.ShapeDtypeStruct((NCORE, NP, 32), _f32),
    mesh=_mesh,
    scratch_types=[
        pltpu.VMEM((CH, 128), _i32),
        pltpu.VMEM((CH, 128), _i32),
        pltpu.VMEM((CH, 128, 32), _f32),
        pltpu.VMEM_SHARED((NP, 32), _f32),
        pltpu.SemaphoreType.DMA,
        pltpu.SemaphoreType.DMA,
    ],
)
def _conv2_sc(src_hbm, dst_hbm, ga_hbm, gb_hbm, z_hbm, out_hbm,
              src_v, dst_v, rows_v, acc_sh, sem_g, sem_s):
    c = lax.axis_index("c")
    s = lax.axis_index("s")

    pltpu.sync_copy(z_hbm.at[pl.ds(s * SLICE, SLICE)],
                    acc_sh.at[pl.ds(s * SLICE, SLICE)])
    plsc.subcore_barrier()

    def run(table_hbm):
        @pl.loop(0, RPS // CH)
        def _(gp):
            base = s * RPS + gp * CH
            pltpu.sync_copy(src_hbm.at[pl.ds(base, CH)], src_v)
            pltpu.sync_copy(dst_hbm.at[pl.ds(base, CH)], dst_v)
            gcps = [pltpu.async_copy(table_hbm.at[src_v.at[j]], rows_v.at[j],
                                     sem_g)
                    for j in range(CH)]
            for cp in gcps:
                cp.wait()
            scps = [pltpu.async_copy(rows_v.at[j], acc_sh.at[dst_v.at[j]],
                                     sem_s, add=True)
                    for j in range(CH)]
            for cp in scps:
                cp.wait()

    @pl.when(c == 0)
    def _():
        run(ga_hbm)

    @pl.when(c == 1)
    def _():
        run(gb_hbm)

    plsc.subcore_barrier()
    pltpu.sync_copy(acc_sh.at[pl.ds(s * SLICE, SLICE)],
                    out_hbm.at[c, pl.ds(s * SLICE, SLICE)])


@functools.partial(
    pl.kernel,
    out_type=(jax.ShapeDtypeStruct((G, 64), _f32),
              jax.ShapeDtypeStruct((G, 64), _f32)),
    mesh=_mesh,
    scratch_types=[
        pltpu.VMEM((C_POOL, 64), _f32),
        pltpu.VMEM((4, 64), _f32),
        pltpu.VMEM((4, 64), _f32),
        pltpu.SMEM((136,), _i32),
    ],
)
def _pool_sc(h_hbm, st_hbm, gmax_hbm, gmean_hbm, buf_v, mx_v, mn_v, st_s):
    c = lax.axis_index("c")
    s = lax.axis_index("s")
    w = c * NSUB + s

    pltpu.sync_copy(st_hbm, st_s)

    for gi in range(4):
        gidx = 4 * w + gi
        s0 = st_s[gidx]
        e0 = st_s[gidx + 1]
        cnt = e0 - s0
        nch = (cnt + (C_POOL - 1)) // C_POOL

        def chunk_body(ci, car):
            start = s0 + ci * C_POOL
            pltpu.sync_copy(h_hbm.at[pl.ds(start, C_POOL)], buf_v)
            rows = jnp.minimum(C_POOL, e0 - start)

            def row_body(r, car2):
                mxs, sms = car2
                nmx, nsm = [], []
                for k in range(4):
                    v = buf_v[pl.ds(r, 1), pl.ds(16 * k, 16)]
                    nmx.append(jnp.maximum(mxs[k], v))
                    nsm.append(sms[k] + v)
                return (tuple(nmx), tuple(nsm))

            return lax.fori_loop(0, rows, row_body, car)

        init = (tuple(jnp.full((1, 16), -jnp.inf, _f32) for _ in range(4)),
                tuple(jnp.zeros((1, 16), _f32) for _ in range(4)))
        mxs, sms = lax.fori_loop(0, nch, chunk_body, init)
        den = jnp.maximum(cnt, 1).astype(_f32)
        for k in range(4):
            mx_v[pl.ds(gi, 1), pl.ds(16 * k, 16)] = mxs[k]
            mn_v[pl.ds(gi, 1), pl.ds(16 * k, 16)] = sms[k] / den

    pltpu.sync_copy(mx_v, gmax_hbm.at[pl.ds(4 * w, 4)])
    pltpu.sync_copy(mn_v, gmean_hbm.at[pl.ds(4 * w, 4)])


def _prep_a_body(batch_ref, we_ref, be_ref, w1_ref, starts_ref, m1_ref):
    b = batch_ref[...]
    for g in range(G + 1):
        starts_ref[g, 0] = jnp.sum((b < g).astype(_i32))
    for g in range(G + 1, 136):
        starts_ref[g, 0] = N
    wcat = jnp.concatenate(
        [we_ref[...], be_ref[...], jnp.zeros((13, 128), _f32)], axis=0)
    m1_ref[...] = jnp.dot(wcat, w1_ref[...],
                          preferred_element_type=_f32,
                          precision=lax.Precision.HIGHEST)


_prep_a = pl.pallas_call(
    _prep_a_body,
    out_shape=(jax.ShapeDtypeStruct((136, 1), _i32),
               jax.ShapeDtypeStruct((16, 64), _f32)),
)


def _prep_b_body(degp_ref, nt_ref, nip_ref, u_ref, dinv_ref):
    deg16 = degp_ref[0] + degp_ref[1] + 1.0
    dinv16 = lax.rsqrt(deg16)
    rows = lax.broadcasted_iota(_i32, (NP, 16), 0)
    mask = (rows < N).astype(_f32)
    xext = jnp.concatenate(
        [nt_ref[...], nip_ref[...], jnp.ones((NP, 1), _f32),
         jnp.zeros((NP, 13), _f32)], axis=1)
    u_ref[...] = dinv16 * xext * mask
    dinv_ref[...] = dinv16[:, :1]


_prep_b = pl.pallas_call(
    _prep_b_body,
    out_shape=(jax.ShapeDtypeStruct((NP, 16), _f32),
               jax.ShapeDtypeStruct((NP, 1), _f32)),
)


def _bpost_body(sp_ref, u_ref, dinv_ref, m1_ref, b1_ref, ga_ref, be_ref,
                g1a_ref, g1b_ref):
    dinv = dinv_ref[...]
    p = dinv * (sp_ref[0] + sp_ref[1] + u_ref[...])
    rows = lax.broadcasted_iota(_i32, (NP, 16), 0)
    p = jnp.where(rows < N, p, 0.0)
    psum = jnp.sum(p, axis=0, keepdims=True)
    pp = lax.dot_general(p, p, (((0,), (0,)), ((), ())),
                         preferred_element_type=_f32,
                         precision=lax.Precision.HIGHEST)
    meanp = psum / N
    cov = pp / N - lax.dot_general(meanp, meanp, (((0,), (0,)), ((), ())),
                                   preferred_element_type=_f32,
                                   precision=lax.Precision.HIGHEST)
    m1 = m1_ref[...]
    mu1 = jnp.dot(meanp, m1, preferred_element_type=_f32,
                  precision=lax.Precision.HIGHEST) + b1_ref[...]
    v = jnp.dot(cov, m1, preferred_element_type=_f32,
                precision=lax.Precision.HIGHEST)
    var1 = jnp.sum(m1 * v, axis=0, keepdims=True)
    istd = lax.rsqrt(var1 + EPS)
    z1 = jnp.dot(p, m1, preferred_element_type=_f32,
                 precision=lax.Precision.HIGHEST) + b1_ref[...]
    h1 = jnp.maximum((z1 - mu1) * istd * ga_ref[...] + be_ref[...], 0.0)
    g1 = dinv * h1
    rows64 = lax.broadcasted_iota(_i32, (NP, 64), 0)
    g1 = jnp.where(rows64 < N, g1, 0.0)
    g1a_ref[...] = g1[:, :32]
    g1b_ref[...] = g1[:, 32:]


_bpost = pl.pallas_call(
    _bpost_body,
    out_shape=(jax.ShapeDtypeStruct((NP, 32), _f32),
               jax.ShapeDtypeStruct((NP, 32), _f32)),
)


def _c1_body(s2_ref, ga_ref, gb_ref, dinv_ref, q_ref, qsum_ref, qq_ref):
    dinv = dinv_ref[...]
    qa = dinv * (s2_ref[0] + ga_ref[...])
    qb = dinv * (s2_ref[1] + gb_ref[...])
    q = jnp.concatenate([qa, qb], axis=1)
    rows64 = lax.broadcasted_iota(_i32, (NP, 64), 0)
    q = jnp.where(rows64 < N, q, 0.0)
    q_ref[...] = q
    qsum_ref[...] = jnp.sum(q, axis=0, keepdims=True)
    qq_ref[...] = lax.dot_general(q, q, (((0,), (0,)), ((), ())),
                                  preferred_element_type=_f32,
                                  precision=lax.Precision.HIGHEST)


_c1 = pl.pallas_call(
    _c1_body,
    out_shape=(jax.ShapeDtypeStruct((NP, 64), _f32),
               jax.ShapeDtypeStruct((1, 64), _f32),
               jax.ShapeDtypeStruct((64, 64), _f32)),
)


def _c2_body(q_ref, w2_ref, b2_ref, qsum_ref, qq_ref, ga_ref, be_ref,
             out_ref):
    w2 = w2_ref[...]
    meanq = qsum_ref[...] / N
    cov = qq_ref[...] / N - lax.dot_general(
        meanq, meanq, (((0,), (0,)), ((), ())),
        preferred_element_type=_f32, precision=lax.Precision.HIGHEST)
    mu2 = jnp.dot(meanq, w2, preferred_element_type=_f32,
                  precision=lax.Precision.HIGHEST) + b2_ref[...]
    v = jnp.dot(cov, w2, preferred_element_type=_f32,
                precision=lax.Precision.HIGHEST)
    var2 = jnp.sum(w2 * v, axis=0, keepdims=True)
    istd = lax.rsqrt(var2 + EPS)
    h2 = jnp.dot(q_ref[...], w2, preferred_element_type=_f32,
                 precision=lax.Precision.HIGHEST) + b2_ref[...]
    h2n = (h2 - mu2) * istd * ga_ref[...] + be_ref[...]
    out_ref[:NP] = h2n
    out_ref[NP:] = jnp.zeros((NP2 - NP, 64), _f32)


_c2 = pl.pallas_call(
    _c2_body,
    out_shape=jax.ShapeDtypeStruct((NP2, 64), _f32),
)


def kernel(node_type, num_inverted_predecessors, edge_index, batch,
           We, be, W1, b1, W2, b2, gamma1, beta1, gamma2, beta2):
    src = edge_index[0].astype(_i32)
    dst = edge_index[1].astype(_i32)
    pad = jnp.full((EPAD - E,), N, _i32)
    src2d = jnp.concatenate([src, pad]).reshape(ROWS, 128)
    dst2d = jnp.concatenate([dst, pad]).reshape(ROWS, 128)
    batch2d = jnp.concatenate(
        [batch.astype(_i32), jnp.full((NP - N,), G, _i32)]).reshape(NB, 128)
    ntf = jnp.pad(node_type.astype(_f32), (0, NP - N)).reshape(NP, 1)
    nipf = jnp.pad(num_inverted_predecessors.astype(_f32),
                   (0, NP - N)).reshape(NP, 1)
    z16 = jnp.zeros((NP, 16), _f32)
    z32 = jnp.zeros((NP, 32), _f32)
    We_ = We.astype(_f32)
    be_ = be.astype(_f32).reshape(1, 128)
    b1_ = b1.astype(_f32).reshape(1, 64)
    ga1 = gamma1.astype(_f32).reshape(1, 64)
    be1 = beta1.astype(_f32).reshape(1, 64)
    b2_ = b2.astype(_f32).reshape(1, 64)
    ga2 = gamma2.astype(_f32).reshape(1, 64)
    be2 = beta2.astype(_f32).reshape(1, 64)

    starts2, m1e = _prep_a(batch2d, We_, be_, W1.astype(_f32))
    degp = _deg_sc(dst2d, z16)
    u16, dinv1 = _prep_b(degp, ntf, nipf)
    sp = _conv1_sc(src2d, dst2d, u16, z16)
    g1a, g1b = _bpost(sp, u16, dinv1, m1e, b1_, ga1, be1)
    s2 = _conv2_sc(src2d, dst2d, g1a, g1b, z32)
    q, qsum, qq = _c1(s2, g1a, g1b, dinv1)
    h2n = _c2(q, W2.astype(_f32), b2_, qsum, qq, ga2, be2)
    starts1 = starts2.reshape(136)
    gmax, gmean = _pool_sc(h2n, starts1)
    return jnp.concatenate([gmax, gmean], axis=1)


# speedup vs baseline: 18.5737x; 18.5737x over previous
import functools

import jax
import jax.numpy as jnp
from jax import lax
from jax.experimental import pallas as pl
from jax.experimental.pallas import tpu as pltpu
from jax.experimental.pallas import tpu_sc as plsc

N = 50000
E = 800000
G = 128
EPS = 1e-5

NP = 50048
NB = 391

NCORE = 2
NSUB = 16
NW = NCORE * NSUB
SLICE = NP // NSUB

EPAD = 819200
ROWS = EPAD // 128
RPW = ROWS // NW
RPS = ROWS // NSUB
CH = 8

C_POOL = 512

_mesh = plsc.VectorSubcoreMesh(
    core_axis_name="c", subcore_axis_name="s", num_cores=NCORE,
    num_subcores=NSUB)

_sc_params = pltpu.CompilerParams(use_tc_tiling_on_sc=False,
                                  needs_layout_passes=False)

_f32 = jnp.float32
_i32 = jnp.int32


@functools.partial(
    pl.kernel,
    out_type=jax.ShapeDtypeStruct((NCORE, NP, 16), _f32),
    mesh=_mesh,
    compiler_params=_sc_params,
    scratch_types=[
        pltpu.VMEM((CH, 128), _i32),
        pltpu.VMEM((128, 16), _f32),
        pltpu.VMEM_SHARED((NP, 16), _f32),
        pltpu.SemaphoreType.DMA,
    ],
)
def _deg_sc(dst_hbm, z_hbm, ones_hbm, out_hbm, idx_v, ones_v, acc_sh, sem):
    c = lax.axis_index("c")
    s = lax.axis_index("s")
    w = c * NSUB + s

    pltpu.sync_copy(ones_hbm, ones_v)
    pltpu.sync_copy(z_hbm.at[pl.ds(s * SLICE, SLICE)],
                    acc_sh.at[pl.ds(s * SLICE, SLICE)])
    plsc.subcore_barrier()

    @pl.loop(0, RPW // CH)
    def _(gp):
        base = w * RPW + gp * CH
        pltpu.sync_copy(dst_hbm.at[pl.ds(base, CH)], idx_v)
        cps = [pltpu.async_copy(ones_v, acc_sh.at[idx_v.at[j]], sem, add=True)
               for j in range(CH)]
        for cp in cps:
            cp.wait()

    plsc.subcore_barrier()
    pltpu.sync_copy(acc_sh.at[pl.ds(s * SLICE, SLICE)],
                    out_hbm.at[c, pl.ds(s * SLICE, SLICE)])


@functools.partial(
    pl.kernel,
    out_type=jax.ShapeDtypeStruct((4, NP, 16), _f32),
    mesh=_mesh,
    compiler_params=_sc_params,
    scratch_types=[
        pltpu.VMEM((CH, 128), _i32),
        pltpu.VMEM((CH, 128), _i32),
        pltpu.VMEM((CH, 128, 16), _f32),
        pltpu.VMEM_SHARED((NP, 16), _f32),
        pltpu.VMEM_SHARED((NP, 16), _f32),
        pltpu.SemaphoreType.DMA,
        pltpu.SemaphoreType.DMA,
    ],
)
def _convw_sc(src_hbm, dst_hbm, g1_hbm, z_hbm, out_hbm,
              src_v, dst_v, rows_v, tab_sh, acc_sh, sem_g, sem_s):
    c = lax.axis_index("c")
    s = lax.axis_index("s")

    for half in range(2):
        quarter = 2 * c + half
        pltpu.sync_copy(z_hbm.at[pl.ds(s * SLICE, SLICE)],
                        acc_sh.at[pl.ds(s * SLICE, SLICE)])
        pltpu.sync_copy(g1_hbm.at[quarter, pl.ds(s * SLICE, SLICE)],
                        tab_sh.at[pl.ds(s * SLICE, SLICE)])
        plsc.subcore_barrier()

        @pl.loop(0, RPS // CH)
        def _(gp):
            base = s * RPS + gp * CH
            pltpu.sync_copy(src_hbm.at[pl.ds(base, CH)], src_v)
            pltpu.sync_copy(dst_hbm.at[pl.ds(base, CH)], dst_v)
            gcps = [pltpu.async_copy(tab_sh.at[src_v.at[j]], rows_v.at[j],
                                     sem_g)
                    for j in range(CH)]
            for cp in gcps:
                cp.wait()
            scps = [pltpu.async_copy(rows_v.at[j], acc_sh.at[dst_v.at[j]],
                                     sem_s, add=True)
                    for j in range(CH)]
            for cp in scps:
                cp.wait()

        plsc.subcore_barrier()
        pltpu.sync_copy(acc_sh.at[pl.ds(s * SLICE, SLICE)],
                        out_hbm.at[quarter, pl.ds(s * SLICE, SLICE)])
        plsc.subcore_barrier()


@functools.partial(
    pl.kernel,
    out_type=(jax.ShapeDtypeStruct((G * 64,), _f32),
              jax.ShapeDtypeStruct((G * 64,), _f32)),
    mesh=_mesh,
    compiler_params=_sc_params,
    scratch_types=[
        pltpu.VMEM((C_POOL * 64,), _f32),
        pltpu.VMEM((256,), _f32),
        pltpu.VMEM((256,), _f32),
        pltpu.VMEM((136,), _i32),
    ],
)
def _pool_sc(h_hbm, st_hbm, gmax_hbm, gmean_hbm, buf_v, mx_v, mn_v, st_v):
    c = lax.axis_index("c")
    s = lax.axis_index("s")

    pltpu.sync_copy(st_hbm, st_v)

    def do_pool(cval):
        w = 2 * s + cval
        stw = st_v[pl.ds(8 * s, 16)]
        lanes = lax.broadcasted_iota(_i32, (16,), 0)

        def ext(k):
            return jnp.max(jnp.where(lanes == k, stw, 0))

        sts = [ext(4 * cval + gi) for gi in range(5)]

        for gi in range(4):
            s0 = sts[gi]
            e0 = sts[gi + 1]
            cnt = e0 - s0
            nch = (cnt + (C_POOL - 1)) // C_POOL

            def chunk_body(ci, car):
                start = s0 + ci * C_POOL
                start_c = jnp.minimum(start, NP - C_POOL)
                d = start - start_c
                pltpu.sync_copy(h_hbm.at[pl.ds(start_c * 64, C_POOL * 64)],
                                buf_v)
                rows = jnp.minimum(C_POOL, e0 - start)

                def row_body(r, car2):
                    mxs, sms = car2
                    nmx, nsm = [], []
                    for k in range(4):
                        v = buf_v[pl.ds((r + d) * 64 + 16 * k, 16)]
                        nmx.append(jnp.maximum(mxs[k], v))
                        nsm.append(sms[k] + v)
                    return (tuple(nmx), tuple(nsm))

                return lax.fori_loop(0, rows, row_body, car)

            init = (tuple(jnp.full((16,), -jnp.inf, _f32) for _ in range(4)),
                    tuple(jnp.zeros((16,), _f32) for _ in range(4)))
            mxs, sms = lax.fori_loop(0, nch, chunk_body, init)
            den = jnp.maximum(cnt, 1).astype(_f32)
            for k in range(4):
                mx_v[pl.ds(gi * 64 + 16 * k, 16)] = mxs[k]
                mn_v[pl.ds(gi * 64 + 16 * k, 16)] = sms[k] / den

        pltpu.sync_copy(mx_v, gmax_hbm.at[pl.ds(w * 256, 256)])
        pltpu.sync_copy(mn_v, gmean_hbm.at[pl.ds(w * 256, 256)])

    @pl.when(c == 0)
    def _():
        do_pool(0)

    @pl.when(c == 1)
    def _():
        do_pool(1)


BR = 3128
NG = NP // BR


def _prep_a_body(batch_ref, starts_ref):
    garr = lax.broadcasted_iota(_i32, (136, 128), 0)

    def body(r, acc):
        row = batch_ref[pl.ds(r, 1), :]
        return acc + (row < garr).astype(_f32)

    acc = lax.fori_loop(0, NB, body, jnp.zeros((136, 128), _f32))
    starts_ref[...] = jnp.sum(acc, axis=1, keepdims=True).astype(_i32)


_prep_a = pl.pallas_call(
    _prep_a_body,
    out_shape=jax.ShapeDtypeStruct((136, 1), _i32),
)


def _mm1_body(x_ref, degp_ref, we_ref, be_ref, w1_ref, gw_ref, dinv_ref):
    h = jnp.dot(x_ref[...], we_ref[...],
                preferred_element_type=_f32) + be_ref[...]
    t = jnp.dot(h, w1_ref[...], preferred_element_type=_f32)
    deg16 = degp_ref[0] + degp_ref[1] + 1.0
    dinv16 = 1.0 / jnp.sqrt(deg16)
    dinv = dinv16[:, :1]
    dinv_ref[...] = dinv
    gid = pl.program_id(0)
    rows = lax.broadcasted_iota(_i32, (BR, 64), 0) + gid * BR
    gw = jnp.where(rows < N, dinv * t, 0.0)
    for k in range(4):
        gw_ref[k] = gw[:, 16 * k:16 * (k + 1)]


_mm1 = pl.pallas_call(
    _mm1_body,
    grid=(NG,),
    in_specs=[
        pl.BlockSpec((BR, 2), lambda i: (i, 0)),
        pl.BlockSpec((2, BR, 16), lambda i: (0, i, 0)),
        pl.BlockSpec((2, 128), lambda i: (0, 0)),
        pl.BlockSpec((1, 128), lambda i: (0, 0)),
        pl.BlockSpec((128, 64), lambda i: (0, 0)),
    ],
    out_specs=[
        pl.BlockSpec((4, BR, 16), lambda i: (0, i, 0)),
        pl.BlockSpec((BR, 1), lambda i: (i, 0)),
    ],
    out_shape=(jax.ShapeDtypeStruct((4, NP, 16), _f32),
               jax.ShapeDtypeStruct((NP, 1), _f32)),
)


def _zstat_body(s_ref, gw_ref, dinv_ref, b_ref, z_ref, zsum_ref, zsq_ref):
    dinv = dinv_ref[...]
    z = jnp.concatenate(
        [dinv * (s_ref[k] + gw_ref[k]) for k in range(4)],
        axis=1) + b_ref[...]
    gid = pl.program_id(0)
    rows = lax.broadcasted_iota(_i32, (BR, 64), 0) + gid * BR
    z = jnp.where(rows < N, z, 0.0)
    z_ref[...] = z

    @pl.when(gid == 0)
    def _():
        zsum_ref[...] = jnp.zeros((1, 64), _f32)
        zsq_ref[...] = jnp.zeros((1, 64), _f32)

    zsum_ref[...] += jnp.sum(z, axis=0, keepdims=True)
    zsq_ref[...] += jnp.sum(z * z, axis=0, keepdims=True)


_zstat = pl.pallas_call(
    _zstat_body,
    grid=(NG,),
    in_specs=[
        pl.BlockSpec((4, BR, 16), lambda i: (0, i, 0)),
        pl.BlockSpec((4, BR, 16), lambda i: (0, i, 0)),
        pl.BlockSpec((BR, 1), lambda i: (i, 0)),
        pl.BlockSpec((1, 64), lambda i: (0, 0)),
    ],
    out_specs=[
        pl.BlockSpec((BR, 64), lambda i: (i, 0)),
        pl.BlockSpec((1, 64), lambda i: (0, 0)),
        pl.BlockSpec((1, 64), lambda i: (0, 0)),
    ],
    out_shape=(jax.ShapeDtypeStruct((NP, 64), _f32),
               jax.ShapeDtypeStruct((1, 64), _f32),
               jax.ShapeDtypeStruct((1, 64), _f32)),
)


def _h1mm_body(z_ref, zsum_ref, zsq_ref, ga_ref, be_ref, dinv_ref, w2_ref,
               gw_ref):
    mu = zsum_ref[...] / N
    var = zsq_ref[...] / N - mu * mu
    istd = lax.rsqrt(var + EPS)
    h1 = jnp.maximum(
        (z_ref[...] - mu) * istd * ga_ref[...] + be_ref[...], 0.0)
    t = jnp.dot(h1, w2_ref[...], preferred_element_type=_f32)
    gid = pl.program_id(0)
    rows = lax.broadcasted_iota(_i32, (BR, 64), 0) + gid * BR
    gw = jnp.where(rows < N, dinv_ref[...] * t, 0.0)
    for k in range(4):
        gw_ref[k] = gw[:, 16 * k:16 * (k + 1)]


_h1mm = pl.pallas_call(
    _h1mm_body,
    grid=(NG,),
    in_specs=[
        pl.BlockSpec((BR, 64), lambda i: (i, 0)),
        pl.BlockSpec((1, 64), lambda i: (0, 0)),
        pl.BlockSpec((1, 64), lambda i: (0, 0)),
        pl.BlockSpec((1, 64), lambda i: (0, 0)),
        pl.BlockSpec((1, 64), lambda i: (0, 0)),
        pl.BlockSpec((BR, 1), lambda i: (i, 0)),
        pl.BlockSpec((64, 64), lambda i: (0, 0)),
    ],
    out_specs=pl.BlockSpec((4, BR, 16), lambda i: (0, i, 0)),
    out_shape=jax.ShapeDtypeStruct((4, NP, 16), _f32),
)


def _c2b_body(h2_ref, hsum_ref, hsq_ref, ga_ref, be_ref, out_ref):
    mu2 = hsum_ref[...] / N
    var2 = hsq_ref[...] / N - mu2 * mu2
    istd = lax.rsqrt(var2 + EPS)
    out_ref[...] = (h2_ref[...] - mu2) * istd * ga_ref[...] + be_ref[...]


_c2b = pl.pallas_call(
    _c2b_body,
    grid=(NG,),
    in_specs=[
        pl.BlockSpec((BR, 64), lambda i: (i, 0)),
        pl.BlockSpec((1, 64), lambda i: (0, 0)),
        pl.BlockSpec((1, 64), lambda i: (0, 0)),
        pl.BlockSpec((1, 64), lambda i: (0, 0)),
        pl.BlockSpec((1, 64), lambda i: (0, 0)),
    ],
    out_specs=pl.BlockSpec((BR, 64), lambda i: (i, 0)),
    out_shape=jax.ShapeDtypeStruct((NP, 64), _f32),
)


def kernel(node_type, num_inverted_predecessors, edge_index, batch,
           We, be, W1, b1, W2, b2, gamma1, beta1, gamma2, beta2):
    src = edge_index[0].astype(_i32)
    dst = edge_index[1].astype(_i32)
    pad = jnp.full((EPAD - E,), N, _i32)
    src2d = jnp.concatenate([src, pad]).reshape(ROWS, 128)
    dst2d = jnp.concatenate([dst, pad]).reshape(ROWS, 128)
    batch2d = jnp.concatenate(
        [batch.astype(_i32), jnp.full((NP - N,), G, _i32)]).reshape(NB, 128)
    x2 = jnp.pad(
        jnp.stack([node_type, num_inverted_predecessors],
                  axis=1).astype(_f32), ((0, NP - N), (0, 0)))
    z16 = jnp.zeros((NP, 16), _f32)
    ones16 = jnp.ones((128, 16), _f32)
    We_ = We.astype(_f32)
    be_ = be.astype(_f32).reshape(1, 128)
    b1_ = b1.astype(_f32).reshape(1, 64)
    ga1 = gamma1.astype(_f32).reshape(1, 64)
    be1 = beta1.astype(_f32).reshape(1, 64)
    b2_ = b2.astype(_f32).reshape(1, 64)
    ga2 = gamma2.astype(_f32).reshape(1, 64)
    be2 = beta2.astype(_f32).reshape(1, 64)

    starts2 = _prep_a(batch2d)
    degp = _deg_sc(dst2d, z16, ones16)
    gw1q, dinv1 = _mm1(x2, degp, We_, be_, W1.astype(_f32))
    s1 = _convw_sc(src2d, dst2d, gw1q, z16)
    z1, zsum, zsq = _zstat(s1, gw1q, dinv1, b1_)
    gw2q = _h1mm(z1, zsum, zsq, ga1, be1, dinv1, W2.astype(_f32))
    s2 = _convw_sc(src2d, dst2d, gw2q, z16)
    h2, hsum, hsq = _zstat(s2, gw2q, dinv1, b2_)
    h2n = _c2b(h2, hsum, hsq, ga2, be2)
    starts1 = starts2.reshape(136)
    gmax, gmean = _pool_sc(h2n.reshape(NP * 64), starts1)
    return jnp.concatenate(
        [gmax.reshape(G, 64), gmean.reshape(G, 64)], axis=1)
